# Initial kernel scaffold; baseline (speedup 1.0000x reference)
import jax
import jax.numpy as jnp
from jax.experimental import pallas as pl


def kernel(x, edge_index, W1, b1, W2, b2):
    raise NotImplementedError("write your pallas kernel here")


# speedup vs baseline: 45.2589x; 45.2589x over previous
import functools

import jax
import jax.numpy as jnp
from jax import lax
from jax.experimental import pallas as pl
from jax.experimental.pallas import tpu as pltpu
from jax.experimental.pallas import tpu_sc as plsc

NC = 2
NS = 16
LANES = 16
NW = NC * NS


def _pad_up(n, m):
    return (n + m - 1) // m * m


def _sc_degree_kernel(chunks, n_pad, rows_per_tile):
    mesh = plsc.VectorSubcoreMesh(core_axis_name="c", subcore_axis_name="s")

    @functools.partial(
        pl.kernel,
        out_type=jax.ShapeDtypeStruct((NC, n_pad), jnp.float32),
        mesh=mesh,
        scratch_types=[
            pltpu.VMEM((chunks, 128), jnp.int32),
            pltpu.VMEM((128,), jnp.float32),
            pltpu.VMEM_SHARED((n_pad,), jnp.float32),
        ],
    )
    def deg_kernel(dst_hbm, zeros_hbm, out_hbm, idx_v, ones_v, deg_sh):
        cid = lax.axis_index("c")
        sid = lax.axis_index("s")
        base = sid * rows_per_tile
        pltpu.sync_copy(
            zeros_hbm.at[pl.ds(base, rows_per_tile)],
            deg_sh.at[pl.ds(base, rows_per_tile)],
        )
        for i in range(128 // LANES):
            ones_v[pl.ds(i * LANES, LANES)] = jnp.ones((LANES,), jnp.float32)
        pltpu.sync_copy(dst_hbm.at[cid, sid], idx_v)
        plsc.subcore_barrier()

        def body(j, carry):
            pltpu.sync_copy(ones_v, deg_sh.at[idx_v.at[j]], add=True)
            return carry

        lax.fori_loop(0, chunks, body, 0)
        plsc.subcore_barrier()
        pltpu.sync_copy(
            deg_sh.at[pl.ds(base, rows_per_tile)],
            out_hbm.at[cid, pl.ds(base, rows_per_tile)],
        )

    return deg_kernel


def _sc_aggregate_kernel(chunks, n_pad, rows_per_tile, d):
    mesh = plsc.VectorSubcoreMesh(core_axis_name="c", subcore_axis_name="s")

    @functools.partial(
        pl.kernel,
        out_type=jax.ShapeDtypeStruct((NC, n_pad, d), jnp.float32),
        mesh=mesh,
        scratch_types=[
            pltpu.VMEM((chunks, 128), jnp.int32),
            pltpu.VMEM((chunks, 128), jnp.int32),
            pltpu.VMEM((128, d), jnp.float32),
            pltpu.VMEM_SHARED((n_pad, d), jnp.float32),
            pltpu.SemaphoreType.DMA,
        ],
        compiler_params=pltpu.CompilerParams(use_tc_tiling_on_sc=False),
    )
    def agg_kernel(src_hbm, dst_hbm, g_hbm, zeros_hbm, out_hbm,
                   sidx_v, didx_v, rows_v, acc_sh, sem):
        cid = lax.axis_index("c")
        sid = lax.axis_index("s")
        base = sid * rows_per_tile
        pltpu.sync_copy(
            zeros_hbm.at[pl.ds(base, rows_per_tile)],
            acc_sh.at[pl.ds(base, rows_per_tile)],
        )
        pltpu.sync_copy(src_hbm.at[cid, sid], sidx_v)
        pltpu.sync_copy(dst_hbm.at[cid, sid], didx_v)
        plsc.subcore_barrier()

        def body(j, carry):
            pltpu.async_copy(g_hbm.at[sidx_v.at[j]], rows_v, sem).wait()
            pltpu.sync_copy(rows_v, acc_sh.at[didx_v.at[j]], add=True)
            return carry

        lax.fori_loop(0, chunks, body, 0)
        plsc.subcore_barrier()
        pltpu.sync_copy(
            acc_sh.at[pl.ds(base, rows_per_tile)],
            out_hbm.at[cid, pl.ds(base, rows_per_tile)],
        )

    return agg_kernel


def _tc_matmul(x, w):
    n, k = x.shape
    d = w.shape[1]
    blk = 2560
    grid = n // blk

    def mm_kernel(x_ref, w_ref, o_ref):
        o_ref[...] = jnp.dot(x_ref[...], w_ref[...],
                             preferred_element_type=jnp.float32)

    return pl.pallas_call(
        mm_kernel,
        grid=(grid,),
        in_specs=[
            pl.BlockSpec((blk, k), lambda i: (i, 0)),
            pl.BlockSpec((k, d), lambda i: (0, 0)),
        ],
        out_specs=pl.BlockSpec((blk, d), lambda i: (i, 0)),
        out_shape=jax.ShapeDtypeStruct((n, d), jnp.float32),
    )(x, w)


def _tc_scale(h, degp):
    n, d = h.shape

    blk = 2560
    grid = n // blk

    def scale_kernel(h_ref, deg_ref, o_ref):
        deg = deg_ref[0, :] + deg_ref[1, :] + 1.0
        dinv = lax.rsqrt(deg)
        o_ref[...] = h_ref[...] * dinv[:, None]

    return pl.pallas_call(
        scale_kernel,
        grid=(grid,),
        in_specs=[
            pl.BlockSpec((blk, d), lambda i: (i, 0)),
            pl.BlockSpec((2, blk), lambda i: (0, i)),
        ],
        out_specs=pl.BlockSpec((blk, d), lambda i: (i, 0)),
        out_shape=jax.ShapeDtypeStruct((n, d), jnp.float32),
    )(h, degp)


def _tc_final(accp, g, degp, b1, w2, b2):
    n, d = g.shape
    blk = 2560
    grid = n // blk

    def fin_kernel(acc_ref, g_ref, deg_ref, b1_ref, w2_ref, b2_ref, o_ref):
        deg = deg_ref[0, :] + deg_ref[1, :] + 1.0
        dinv = lax.rsqrt(deg)
        tot = acc_ref[0] + acc_ref[1] + g_ref[...]
        agg = tot * dinv[:, None] + b1_ref[0, :]
        h1 = jnp.maximum(agg, 0.0)
        o_ref[...] = (
            jnp.dot(h1, w2_ref[...], preferred_element_type=jnp.float32)
            + b2_ref[0, :]
        )

    return pl.pallas_call(
        fin_kernel,
        grid=(grid,),
        in_specs=[
            pl.BlockSpec((2, blk, d), lambda i: (0, i, 0)),
            pl.BlockSpec((blk, d), lambda i: (i, 0)),
            pl.BlockSpec((2, blk), lambda i: (0, i)),
            pl.BlockSpec((1, d), lambda i: (0, 0)),
            pl.BlockSpec((d, d), lambda i: (0, 0)),
            pl.BlockSpec((1, d), lambda i: (0, 0)),
        ],
        out_specs=pl.BlockSpec((blk, d), lambda i: (i, 0)),
        out_shape=jax.ShapeDtypeStruct((n, d), jnp.float32),
    )(accp, g, degp, b1, w2, b2)


def kernel(x, edge_index, W1, b1, W2, b2):
    n, k_in = x.shape
    d = W1.shape[1]
    e = edge_index.shape[1]

    n_pad = _pad_up(n + 1, 128 * NS)
    rows_per_tile = n_pad // NS
    chunks = _pad_up(e, NW * 128) // (NW * 128)
    e_pad = NW * chunks * 128

    src = edge_index[0].astype(jnp.int32)
    dst = edge_index[1].astype(jnp.int32)
    pad = e_pad - e
    src_r = jnp.concatenate([src, jnp.zeros((pad,), jnp.int32)])
    dst_r = jnp.concatenate([dst, jnp.full((pad,), n, jnp.int32)])
    src_r = src_r.reshape(NC, NS, chunks, 128)
    dst_r = dst_r.reshape(NC, NS, chunks, 128)

    zeros1 = jnp.zeros((n_pad,), jnp.float32)
    zeros2 = jnp.zeros((n_pad, d), jnp.float32)
    xp = jnp.pad(x, ((0, n_pad - n), (0, 0)))

    degp = _sc_degree_kernel(chunks, n_pad, rows_per_tile)(dst_r, zeros1)

    h = _tc_matmul(xp, W1)
    g = _tc_scale(h, degp)

    accp = _sc_aggregate_kernel(chunks, n_pad, rows_per_tile, d)(
        src_r, dst_r, g, zeros2)

    out = _tc_final(accp, g, degp, b1.reshape(1, d), W2, b2.reshape(1, d))
    return out[:n]


# speedup vs baseline: 49.5277x; 1.0943x over previous
import functools

import jax
import jax.numpy as jnp
from jax import lax
from jax.experimental import pallas as pl
from jax.experimental.pallas import tpu as pltpu
from jax.experimental.pallas import tpu_sc as plsc

NC = 2
NS = 16
LANES = 16
NW = NC * NS


def _pad_up(n, m):
    return (n + m - 1) // m * m


def _sc_degree_kernel(chunks, n_pad, rows_per_tile):
    mesh = plsc.VectorSubcoreMesh(core_axis_name="c", subcore_axis_name="s")

    @functools.partial(
        pl.kernel,
        out_type=jax.ShapeDtypeStruct((NC, n_pad), jnp.float32),
        mesh=mesh,
        scratch_types=[
            pltpu.VMEM((chunks * 128,), jnp.int32),
            pltpu.VMEM((chunks * 128,), jnp.float32),
            pltpu.VMEM_SHARED((n_pad,), jnp.float32),
        ],
    )
    def deg_kernel(dst_hbm, ones_hbm, zeros_hbm, out_hbm, idx_v, ones_v,
                   deg_sh):
        cid = lax.axis_index("c")
        sid = lax.axis_index("s")
        base = sid * rows_per_tile
        pltpu.sync_copy(
            zeros_hbm.at[pl.ds(base, rows_per_tile)],
            deg_sh.at[pl.ds(base, rows_per_tile)],
        )
        pltpu.sync_copy(ones_hbm, ones_v)
        pltpu.sync_copy(dst_hbm.at[cid, sid], idx_v)
        plsc.subcore_barrier()
        pltpu.sync_copy(ones_v, deg_sh.at[idx_v], add=True)
        plsc.subcore_barrier()
        pltpu.sync_copy(
            deg_sh.at[pl.ds(base, rows_per_tile)],
            out_hbm.at[cid, pl.ds(base, rows_per_tile)],
        )

    return deg_kernel


def _sc_aggregate_kernel(n_mega, k2, n_pad, rows_per_tile, d):
    mesh = plsc.VectorSubcoreMesh(core_axis_name="c", subcore_axis_name="s")

    @functools.partial(
        pl.kernel,
        out_type=jax.ShapeDtypeStruct((NC, n_pad, d), jnp.float32),
        mesh=mesh,
        scratch_types=[
            pltpu.VMEM((n_mega, k2 * 128), jnp.int32),
            pltpu.VMEM((n_mega, k2 * 128), jnp.int32),
            pltpu.VMEM((k2 * 128, d), jnp.float32),
            pltpu.VMEM((k2 * 128, d), jnp.float32),
            pltpu.VMEM_SHARED((n_pad, d), jnp.float32),
            pltpu.SemaphoreType.DMA,
            pltpu.SemaphoreType.DMA,
            pltpu.SemaphoreType.DMA,
            pltpu.SemaphoreType.DMA,
        ],
        compiler_params=pltpu.CompilerParams(use_tc_tiling_on_sc=False),
    )
    def agg_kernel(src_hbm, dst_hbm, g_hbm, zeros_hbm, out_hbm,
                   sidx_v, didx_v, rows0_v, rows1_v, acc_sh,
                   sg0, sg1, ss0, ss1):
        cid = lax.axis_index("c")
        sid = lax.axis_index("s")
        base = sid * rows_per_tile
        pltpu.sync_copy(
            zeros_hbm.at[pl.ds(base, rows_per_tile)],
            acc_sh.at[pl.ds(base, rows_per_tile)],
        )
        pltpu.sync_copy(src_hbm.at[cid, sid], sidx_v)
        pltpu.sync_copy(dst_hbm.at[cid, sid], didx_v)
        plsc.subcore_barrier()

        rows = (rows0_v, rows1_v)
        sgs = (sg0, sg1)
        sss = (ss0, ss1)

        gat = [None, None]
        sca = [None, None]
        gat[0] = pltpu.async_copy(g_hbm.at[sidx_v.at[0]], rows[0], sgs[0])
        for m in range(n_mega):
            b = m & 1
            if m + 1 < n_mega:
                b2 = (m + 1) & 1
                if sca[b2] is not None:
                    sca[b2].wait()
                gat[b2] = pltpu.async_copy(
                    g_hbm.at[sidx_v.at[m + 1]], rows[b2], sgs[b2])
            gat[b].wait()
            sca[b] = pltpu.async_copy(
                rows[b], acc_sh.at[didx_v.at[m]], sss[b], add=True)
        for b in range(2):
            if sca[b] is not None:
                sca[b].wait()
        plsc.subcore_barrier()
        pltpu.sync_copy(
            acc_sh.at[pl.ds(base, rows_per_tile)],
            out_hbm.at[cid, pl.ds(base, rows_per_tile)],
        )

    return agg_kernel


def _tc_matmul(x, w):
    n, k = x.shape
    d = w.shape[1]
    blk = 2560
    grid = n // blk

    def mm_kernel(x_ref, w_ref, o_ref):
        o_ref[...] = jnp.dot(x_ref[...], w_ref[...],
                             preferred_element_type=jnp.float32)

    return pl.pallas_call(
        mm_kernel,
        grid=(grid,),
        in_specs=[
            pl.BlockSpec((blk, k), lambda i: (i, 0)),
            pl.BlockSpec((k, d), lambda i: (0, 0)),
        ],
        out_specs=pl.BlockSpec((blk, d), lambda i: (i, 0)),
        out_shape=jax.ShapeDtypeStruct((n, d), jnp.float32),
    )(x, w)


def _tc_scale(h, degp):
    n, d = h.shape

    blk = 2560
    grid = n // blk

    def scale_kernel(h_ref, deg_ref, o_ref):
        deg = deg_ref[0, :] + deg_ref[1, :] + 1.0
        dinv = lax.rsqrt(deg)
        o_ref[...] = h_ref[...] * dinv[:, None]

    return pl.pallas_call(
        scale_kernel,
        grid=(grid,),
        in_specs=[
            pl.BlockSpec((blk, d), lambda i: (i, 0)),
            pl.BlockSpec((2, blk), lambda i: (0, i)),
        ],
        out_specs=pl.BlockSpec((blk, d), lambda i: (i, 0)),
        out_shape=jax.ShapeDtypeStruct((n, d), jnp.float32),
    )(h, degp)


def _tc_final(accp, g, degp, b1, w2, b2):
    n, d = g.shape
    blk = 2560
    grid = n // blk

    def fin_kernel(acc_ref, g_ref, deg_ref, b1_ref, w2_ref, b2_ref, o_ref):
        deg = deg_ref[0, :] + deg_ref[1, :] + 1.0
        dinv = lax.rsqrt(deg)
        tot = acc_ref[0] + acc_ref[1] + g_ref[...]
        agg = tot * dinv[:, None] + b1_ref[0, :]
        h1 = jnp.maximum(agg, 0.0)
        o_ref[...] = (
            jnp.dot(h1, w2_ref[...], preferred_element_type=jnp.float32)
            + b2_ref[0, :]
        )

    return pl.pallas_call(
        fin_kernel,
        grid=(grid,),
        in_specs=[
            pl.BlockSpec((2, blk, d), lambda i: (0, i, 0)),
            pl.BlockSpec((blk, d), lambda i: (i, 0)),
            pl.BlockSpec((2, blk), lambda i: (0, i)),
            pl.BlockSpec((1, d), lambda i: (0, 0)),
            pl.BlockSpec((d, d), lambda i: (0, 0)),
            pl.BlockSpec((1, d), lambda i: (0, 0)),
        ],
        out_specs=pl.BlockSpec((blk, d), lambda i: (i, 0)),
        out_shape=jax.ShapeDtypeStruct((n, d), jnp.float32),
    )(accp, g, degp, b1, w2, b2)


def kernel(x, edge_index, W1, b1, W2, b2):
    n, k_in = x.shape
    d = W1.shape[1]
    e = edge_index.shape[1]

    n_pad = _pad_up(n + 1, 128 * NS)
    rows_per_tile = n_pad // NS
    k2 = 16
    chunks = _pad_up(_pad_up(e, NW * 128) // (NW * 128), k2)
    n_mega = chunks // k2
    e_pad = NW * chunks * 128

    src = edge_index[0].astype(jnp.int32)
    dst = edge_index[1].astype(jnp.int32)
    pad = e_pad - e
    src_r = jnp.concatenate([src, jnp.zeros((pad,), jnp.int32)])
    dst_r = jnp.concatenate([dst, jnp.full((pad,), n, jnp.int32)])
    src_r = src_r.reshape(NC, NS, n_mega, k2 * 128)
    dst_r = dst_r.reshape(NC, NS, n_mega, k2 * 128)
    dst_flat = dst_r.reshape(NC, NS, chunks * 128)

    ones2d = jnp.ones((chunks * 128,), jnp.float32)
    zeros1 = jnp.zeros((n_pad,), jnp.float32)
    zeros2 = jnp.zeros((n_pad, d), jnp.float32)
    xp = jnp.pad(x, ((0, n_pad - n), (0, 0)))

    degp = _sc_degree_kernel(chunks, n_pad, rows_per_tile)(
        dst_flat, ones2d, zeros1)

    h = _tc_matmul(xp, W1)
    g = _tc_scale(h, degp)

    accp = _sc_aggregate_kernel(n_mega, k2, n_pad, rows_per_tile, d)(
        src_r, dst_r, g, zeros2)

    out = _tc_final(accp, g, degp, b1.reshape(1, d), W2, b2.reshape(1, d))
    return out[:n]


# speedup vs baseline: 72.9110x; 1.4721x over previous
import functools

import jax
import jax.numpy as jnp
from jax import lax
from jax.experimental import pallas as pl
from jax.experimental.pallas import tpu as pltpu
from jax.experimental.pallas import tpu_sc as plsc

NC = 2
NS = 16
LANES = 16
NW = NC * NS


def _pad_up(n, m):
    return (n + m - 1) // m * m


def _sc_degree_kernel(ept, n_pad, rows_per_tile):
    mesh = plsc.VectorSubcoreMesh(core_axis_name="c", subcore_axis_name="s")

    @functools.partial(
        pl.kernel,
        out_type=jax.ShapeDtypeStruct((NC, n_pad), jnp.float32),
        mesh=mesh,
        scratch_types=[
            pltpu.VMEM((ept,), jnp.int32),
            pltpu.VMEM((ept,), jnp.float32),
            pltpu.VMEM_SHARED((n_pad,), jnp.float32),
        ],
    )
    def deg_kernel(dst_hbm, ones_hbm, zeros_hbm, out_hbm, idx_v, ones_v,
                   deg_sh):
        cid = lax.axis_index("c")
        sid = lax.axis_index("s")
        base = sid * rows_per_tile
        pltpu.sync_copy(
            zeros_hbm.at[pl.ds(base, rows_per_tile)],
            deg_sh.at[pl.ds(base, rows_per_tile)],
        )
        pltpu.sync_copy(ones_hbm, ones_v)
        pltpu.sync_copy(dst_hbm.at[cid, sid], idx_v)
        plsc.subcore_barrier()
        pltpu.sync_copy(ones_v, deg_sh.at[idx_v], add=True)
        plsc.subcore_barrier()
        pltpu.sync_copy(
            deg_sh.at[pl.ds(base, rows_per_tile)],
            out_hbm.at[cid, pl.ds(base, rows_per_tile)],
        )

    return deg_kernel


def _sc_aggregate_kernel(n_mega, chunk, n_pad, rows_per_tile, d):
    mesh = plsc.VectorSubcoreMesh(core_axis_name="c", subcore_axis_name="s")

    @functools.partial(
        pl.kernel,
        out_type=jax.ShapeDtypeStruct((NC, n_pad, d), jnp.float32),
        mesh=mesh,
        scratch_types=[
            pltpu.VMEM((n_mega, chunk), jnp.int32),
            pltpu.VMEM((n_mega, chunk), jnp.int32),
            pltpu.VMEM((chunk, d), jnp.float32),
            pltpu.VMEM((chunk, d), jnp.float32),
            pltpu.VMEM_SHARED((n_pad, d), jnp.float32),
            pltpu.SemaphoreType.DMA,
            pltpu.SemaphoreType.DMA,
            pltpu.SemaphoreType.DMA,
            pltpu.SemaphoreType.DMA,
        ],
        compiler_params=pltpu.CompilerParams(use_tc_tiling_on_sc=False),
    )
    def agg_kernel(src_hbm, dst_hbm, g_hbm, zeros_hbm, out_hbm,
                   sidx_v, didx_v, rows0_v, rows1_v, acc_sh,
                   sg0, sg1, ss0, ss1):
        cid = lax.axis_index("c")
        sid = lax.axis_index("s")
        base = sid * rows_per_tile
        pltpu.sync_copy(
            zeros_hbm.at[pl.ds(base, rows_per_tile)],
            acc_sh.at[pl.ds(base, rows_per_tile)],
        )
        pltpu.sync_copy(src_hbm.at[cid, sid], sidx_v)
        pltpu.sync_copy(dst_hbm.at[cid, sid], didx_v)
        plsc.subcore_barrier()

        rows = (rows0_v, rows1_v)
        sgs = (sg0, sg1)
        sss = (ss0, ss1)

        gat = [None, None]
        sca = [None, None]
        gat[0] = pltpu.async_copy(g_hbm.at[sidx_v.at[0]], rows[0], sgs[0])
        for m in range(n_mega):
            b = m & 1
            if m + 1 < n_mega:
                b2 = (m + 1) & 1
                if sca[b2] is not None:
                    sca[b2].wait()
                gat[b2] = pltpu.async_copy(
                    g_hbm.at[sidx_v.at[m + 1]], rows[b2], sgs[b2])
            gat[b].wait()
            sca[b] = pltpu.async_copy(
                rows[b], acc_sh.at[didx_v.at[m]], sss[b], add=True)
        for b in range(2):
            if sca[b] is not None:
                sca[b].wait()
        plsc.subcore_barrier()
        pltpu.sync_copy(
            acc_sh.at[pl.ds(base, rows_per_tile)],
            out_hbm.at[cid, pl.ds(base, rows_per_tile)],
        )

    return agg_kernel


def _tc_matmul(x, w):
    n, k = x.shape
    d = w.shape[1]
    blk = 2560
    grid = n // blk

    def mm_kernel(x_ref, w_ref, o_ref):
        o_ref[...] = jnp.dot(x_ref[...], w_ref[...],
                             preferred_element_type=jnp.float32)

    return pl.pallas_call(
        mm_kernel,
        grid=(grid,),
        in_specs=[
            pl.BlockSpec((blk, k), lambda i: (i, 0)),
            pl.BlockSpec((k, d), lambda i: (0, 0)),
        ],
        out_specs=pl.BlockSpec((blk, d), lambda i: (i, 0)),
        out_shape=jax.ShapeDtypeStruct((n, d), jnp.float32),
    )(x, w)


def _tc_scale(h, degp):
    n, d = h.shape

    blk = 2560
    grid = n // blk

    def scale_kernel(h_ref, deg_ref, o_ref):
        deg = deg_ref[0, :] + deg_ref[1, :] + 1.0
        dinv = lax.rsqrt(deg)
        o_ref[...] = h_ref[...] * dinv[:, None]

    return pl.pallas_call(
        scale_kernel,
        grid=(grid,),
        in_specs=[
            pl.BlockSpec((blk, d), lambda i: (i, 0)),
            pl.BlockSpec((2, blk), lambda i: (0, i)),
        ],
        out_specs=pl.BlockSpec((blk, d), lambda i: (i, 0)),
        out_shape=jax.ShapeDtypeStruct((n, d), jnp.float32),
    )(h, degp)


def _tc_final(accp, g, degp, b1, w2, b2):
    n, d = g.shape
    blk = 2560
    grid = n // blk

    def fin_kernel(acc_ref, g_ref, deg_ref, b1_ref, w2_ref, b2_ref, o_ref):
        deg = deg_ref[0, :] + deg_ref[1, :] + 1.0
        dinv = lax.rsqrt(deg)
        tot = acc_ref[0] + acc_ref[1] + g_ref[...]
        agg = tot * dinv[:, None] + b1_ref[0, :]
        h1 = jnp.maximum(agg, 0.0)
        o_ref[...] = (
            jnp.dot(h1, w2_ref[...], preferred_element_type=jnp.float32)
            + b2_ref[0, :]
        )

    return pl.pallas_call(
        fin_kernel,
        grid=(grid,),
        in_specs=[
            pl.BlockSpec((2, blk, d), lambda i: (0, i, 0)),
            pl.BlockSpec((blk, d), lambda i: (i, 0)),
            pl.BlockSpec((2, blk), lambda i: (0, i)),
            pl.BlockSpec((1, d), lambda i: (0, 0)),
            pl.BlockSpec((d, d), lambda i: (0, 0)),
            pl.BlockSpec((1, d), lambda i: (0, 0)),
        ],
        out_specs=pl.BlockSpec((blk, d), lambda i: (i, 0)),
        out_shape=jax.ShapeDtypeStruct((n, d), jnp.float32),
    )(accp, g, degp, b1, w2, b2)


def kernel(x, edge_index, W1, b1, W2, b2):
    n, k_in = x.shape
    d = W1.shape[1]
    e = edge_index.shape[1]

    n_pad = _pad_up(n + 1, 128 * NS)
    rows_per_tile = n_pad // NS
    ept_raw = -(-e // NW)
    n_mega = -(-ept_raw // 2048)
    chunk = _pad_up(-(-ept_raw // n_mega), 8)
    ept = n_mega * chunk
    e_pad = NW * ept
    pad = e_pad - e

    src = edge_index[0].astype(jnp.int32)
    dst = edge_index[1].astype(jnp.int32)
    if pad:
        trash = n + jnp.arange(pad, dtype=jnp.int32) % (n_pad - n)
        src = jnp.concatenate([src, jnp.zeros((pad,), jnp.int32)])
        dst = jnp.concatenate([dst, trash])
    src_r = src.reshape(NC, NS, n_mega, chunk)
    dst_r = dst.reshape(NC, NS, n_mega, chunk)
    dst_flat = dst_r.reshape(NC, NS, ept)

    ones2d = jnp.ones((ept,), jnp.float32)
    zeros1 = jnp.zeros((n_pad,), jnp.float32)
    zeros2 = jnp.zeros((n_pad, d), jnp.float32)
    xp = jnp.pad(x, ((0, n_pad - n), (0, 0)))

    degp = _sc_degree_kernel(ept, n_pad, rows_per_tile)(
        dst_flat, ones2d, zeros1)

    h = _tc_matmul(xp, W1)
    g = _tc_scale(h, degp)

    accp = _sc_aggregate_kernel(n_mega, chunk, n_pad, rows_per_tile, d)(
        src_r, dst_r, g, zeros2)

    out = _tc_final(accp, g, degp, b1.reshape(1, d), W2, b2.reshape(1, d))
    return out[:n]
